# baseline (speedup 1.0000x reference)
import functools

import jax
import jax.numpy as jnp
from jax.experimental import pallas as pl
from jax.experimental.pallas import tpu as pltpu

_MIB = 1024 * 1024


def _schedule(B, bt):
    ramp = [s for s in (2, 2, 4) if s < bt]
    body = B - 2 * sum(ramp)
    assert body % bt == 0
    return ramp + [bt] * (body // bt) + ramp[::-1]


def _se_pipe(x_hbm, w1_ref, b1_ref, w2_ref, b2_ref, o_hbm,
             x_buf, o_buf, in_sem, out_sem, *, chunks, inv_hw):
    n = len(chunks)
    starts = [sum(chunks[:i]) for i in range(n)]

    def dma_in(i):
        slot = i % 2
        pltpu.make_async_copy(
            x_hbm.at[pl.ds(starts[i], chunks[i])],
            x_buf.at[slot, pl.ds(0, chunks[i])], in_sem.at[slot]).start()

    def wait_in(i):
        slot = i % 2
        pltpu.make_async_copy(
            x_buf.at[slot, pl.ds(0, chunks[i])],
            x_buf.at[slot, pl.ds(0, chunks[i])], in_sem.at[slot]).wait()

    def dma_out(i):
        slot = i % 2
        pltpu.make_async_copy(
            o_buf.at[slot, pl.ds(0, chunks[i])],
            o_hbm.at[pl.ds(starts[i], chunks[i])], out_sem.at[slot]).start()

    def wait_out(i):
        slot = i % 2
        pltpu.make_async_copy(
            o_buf.at[slot, pl.ds(0, chunks[i])],
            o_buf.at[slot, pl.ds(0, chunks[i])], out_sem.at[slot]).wait()

    dma_in(0)
    for i in range(n):
        if i + 1 < n:
            dma_in(i + 1)
        wait_in(i)
        if i >= 2:
            wait_out(i - 2)
        bt_i = chunks[i]
        xb = x_buf[i % 2, pl.ds(0, bt_i)]
        s = jnp.sum(xb, axis=2, dtype=jnp.float32) * inv_hw
        h = jax.lax.dot_general(s, w1_ref[...], (((1,), (1,)), ((), ())),
                                preferred_element_type=jnp.float32)
        h = jnp.maximum(h + b1_ref[...], 0.0)
        g = jax.lax.dot_general(h, w2_ref[...], (((1,), (1,)), ((), ())),
                                preferred_element_type=jnp.float32)
        g = jax.nn.sigmoid(g + b2_ref[...])
        o_buf[i % 2, pl.ds(0, bt_i)] = xb * g[:, :, None]
        dma_out(i)
    wait_out(n - 2)
    wait_out(n - 1)


@jax.jit
def kernel(x, w1, b1, w2, b2):
    B, C, H, W = x.shape
    Cr = w1.shape[0]
    HW = H * W
    f32 = jnp.float32

    x3 = x.reshape(B, C, HW)
    b1r = b1.reshape(1, Cr).astype(f32)
    b2r = b2.reshape(1, C).astype(f32)

    bt = 8
    chunks = _schedule(B, bt)
    buf_bytes = 2 * 2 * bt * C * HW * jnp.dtype(x.dtype).itemsize

    out = pl.pallas_call(
        functools.partial(_se_pipe, chunks=chunks, inv_hw=1.0 / HW),
        out_shape=jax.ShapeDtypeStruct((B, C, HW), x.dtype),
        in_specs=[
            pl.BlockSpec(memory_space=pl.ANY),
            pl.BlockSpec(memory_space=pltpu.VMEM),
            pl.BlockSpec(memory_space=pltpu.VMEM),
            pl.BlockSpec(memory_space=pltpu.VMEM),
            pl.BlockSpec(memory_space=pltpu.VMEM),
        ],
        out_specs=pl.BlockSpec(memory_space=pl.ANY),
        scratch_shapes=[
            pltpu.VMEM((2, bt, C, HW), x.dtype),
            pltpu.VMEM((2, bt, C, HW), x.dtype),
            pltpu.SemaphoreType.DMA((2,)),
            pltpu.SemaphoreType.DMA((2,)),
        ],
        compiler_params=pltpu.CompilerParams(
            vmem_limit_bytes=buf_bytes + 8 * _MIB,
        ),
    )(x3, w1.astype(f32), b1r, w2.astype(f32), b2r)
    return out.reshape(B, C, H, W)
